# baseline (speedup 1.0000x reference)
import jax
import jax.numpy as jnp
from jax import lax
from jax.experimental import pallas as pl
from jax.experimental.pallas import tpu as pltpu
from jax.experimental.pallas import tpu_sc as plsc

N = 10000
E = 320000
F_IN = 128
H = 4
F_OUT = 32
HF = H * F_OUT
CW = HF + 16

NC = 2
NS = 16
NW = NC * NS
L = 16
EW = E // NW
CH = 80
NCHUNK = EW // CH
ROWS_PER_TILE = N // NS

f32 = jnp.float32
i32 = jnp.int32


def _full16(v):
    return jnp.full((L,), v, dtype=i32)


def _tc1_body(x_ref, wpt_ref, sel_ref, proj_ref, stab_ref, mhat_ref):
    xb = x_ref[...]
    p = jnp.dot(xb, wpt_ref[...], preferred_element_type=f32)
    proj_ref[...] = p
    st = jnp.dot(p, sel_ref[...], preferred_element_type=f32)
    stab_ref[...] = st
    m0 = jnp.max(st[:, 0:H]) + jnp.max(st[:, H:2 * H])
    mhat = jnp.maximum(m0, 0.2 * m0)
    mhat_ref[...] = jnp.full((8, 128), mhat, dtype=f32)


def _tc1(x, wpt, sel):
    return pl.pallas_call(
        _tc1_body,
        out_shape=(
            jax.ShapeDtypeStruct((N, HF), f32),
            jax.ShapeDtypeStruct((N, 16), f32),
            jax.ShapeDtypeStruct((8, 128), f32),
        ),
    )(x, wpt, sel)


def _k1_body(tab_hbm, mhat_hbm, ei_hbm, rel_hbm, zeros_hbm,
             exps_hbm, cprt_hbm,
             mhat_v, sidx, tidx, combo, sbuf, tbuf, expbuf,
             isem, gsem, rsem, esem, ssem,
             combo_sh):
    c = lax.axis_index("c")
    s = lax.axis_index("s")
    wid = c * NS + s
    ebase = wid * EW
    rbase = s * ROWS_PER_TILE
    row0 = wid * NCHUNK

    pltpu.sync_copy(zeros_hbm, combo_sh.at[pl.ds(rbase, ROWS_PER_TILE)])
    pltpu.sync_copy(mhat_hbm.at[0, pl.ds(0, L)], mhat_v)

    def _zrow(r, carry):
        for b in range(3):
            combo[b, r, pl.ds(HF, 16)] = jnp.zeros((L,), dtype=f32)
        return carry
    lax.fori_loop(0, CH, _zrow, 0)

    plsc.subcore_barrier()

    mhat_vec = mhat_v[:]
    iota16 = lax.iota(i32, L)

    def _fire_idx(j):
        b4 = lax.rem(j, 4)
        pltpu.async_copy(ei_hbm.at[0, row0 + j], sidx.at[b4], isem)
        pltpu.async_copy(ei_hbm.at[1, row0 + j], tidx.at[b4], isem)

    def _drain_idx():
        pltpu.make_async_copy(ei_hbm.at[0, 0], sidx.at[0], isem).wait()
        pltpu.make_async_copy(ei_hbm.at[1, 0], tidx.at[0], isem).wait()

    def _fire_gathers(j):
        b2 = lax.rem(j, 2)
        b3 = lax.rem(j, 3)
        b4 = lax.rem(j, 4)
        pltpu.async_copy(tab_hbm.at[sidx.at[b4]], sbuf.at[b2], gsem)
        pltpu.async_copy(tab_hbm.at[tidx.at[b4]], tbuf.at[b2], gsem)
        pltpu.async_copy(rel_hbm.at[pl.ds(ebase + j * CH, CH)],
                         combo.at[b3, :, pl.ds(0, HF)], rsem)

    def _drain_gathers():
        pltpu.make_async_copy(tab_hbm.at[sidx.at[0]], sbuf.at[0], gsem).wait()
        pltpu.make_async_copy(tab_hbm.at[tidx.at[0]], tbuf.at[0], gsem).wait()
        pltpu.make_async_copy(rel_hbm.at[pl.ds(0, CH)],
                              combo.at[0, :, pl.ds(0, HF)], rsem).wait()

    def _drain_scatter():
        pltpu.make_async_copy(combo.at[0], combo_sh.at[tidx.at[0]],
                              ssem).wait()

    def _drain_expw():
        pltpu.make_async_copy(expbuf.at[0],
                              exps_hbm.at[:, pl.ds(0, CH)], esem).wait()

    _fire_idx(0)
    _fire_idx(1)
    _drain_idx()
    _fire_gathers(0)

    def _chunk(j, carry):
        @pl.when(j >= 2)
        def _():
            _drain_scatter()
            _drain_expw()

        @pl.when(j < NCHUNK - 2)
        def _():
            _fire_idx(j + 2)

        @pl.when(j < NCHUNK - 1)
        def _():
            _drain_idx()
            _fire_gathers(j + 1)

        _drain_gathers()

        b2 = lax.rem(j, 2)
        b3 = lax.rem(j, 3)
        b5 = lax.rem(j, 4)
        for g in range(CH // L):
            rows16 = g * L + iota16
            for h in range(H):
                a = plsc.load_gather(sbuf.at[b2], [rows16, _full16(h)])
                b = plsc.load_gather(tbuf.at[b2], [rows16, _full16(H + h)])
                sc = a + b
                sc = jnp.maximum(sc, 0.2 * sc) - mhat_vec
                ex = jnp.exp(sc)
                plsc.store_scatter(combo.at[b3], [rows16, _full16(HF + h)],
                                   ex)
                expbuf[b2, h, pl.ds(g * L, L)] = ex
        pltpu.async_copy(expbuf.at[b2],
                         exps_hbm.at[:, pl.ds(ebase + j * CH, CH)], esem)
        pltpu.async_copy(combo.at[b3], combo_sh.at[tidx.at[b5]], ssem,
                         add=True)
        return carry

    lax.fori_loop(0, NCHUNK, _chunk, 0)

    _drain_scatter()
    _drain_scatter()
    _drain_expw()
    _drain_expw()

    plsc.subcore_barrier()

    pltpu.sync_copy(combo_sh.at[pl.ds(rbase, ROWS_PER_TILE)],
                    cprt_hbm.at[c, pl.ds(rbase, ROWS_PER_TILE)])


def _k1(stab, mhat, ei3, rel, zeros):
    kfn = pl.kernel(
        _k1_body,
        out_type=(
            jax.ShapeDtypeStruct((H, E), f32),
            jax.ShapeDtypeStruct((NC, N, CW), f32),
        ),
        mesh=plsc.VectorSubcoreMesh(core_axis_name="c", subcore_axis_name="s"),
        compiler_params=pltpu.CompilerParams(use_tc_tiling_on_sc=False,
                                             needs_layout_passes=False),
        scratch_types=[
            pltpu.VMEM((L,), f32),
            pltpu.VMEM((4, CH), i32),
            pltpu.VMEM((4, CH), i32),
            pltpu.VMEM((3, CH, CW), f32),
            pltpu.VMEM((2, CH, 16), f32),
            pltpu.VMEM((2, CH, 16), f32),
            pltpu.VMEM((2, H, CH), f32),
            pltpu.SemaphoreType.DMA,
            pltpu.SemaphoreType.DMA,
            pltpu.SemaphoreType.DMA,
            pltpu.SemaphoreType.DMA,
            pltpu.SemaphoreType.DMA,
            pltpu.VMEM_SHARED((N, CW), f32),
        ],
    )
    return kfn(stab, mhat, ei3, rel, zeros)


def _k2_body(exps_hbm, ei_hbm, proj_hbm, zeros_hbm,
             aprt_hbm,
             sidx, tidx, projbuf, attnbuf,
             isem, gsem, esem, ssem,
             acc_sh):
    c = lax.axis_index("c")
    s = lax.axis_index("s")
    wid = c * NS + s
    ebase = wid * EW
    rbase = s * ROWS_PER_TILE
    row0 = wid * NCHUNK

    pltpu.sync_copy(zeros_hbm.at[:, pl.ds(0, HF)],
                    acc_sh.at[pl.ds(rbase, ROWS_PER_TILE)])

    plsc.subcore_barrier()

    iota16 = lax.iota(i32, L)

    def _fire_idx(j):
        b4 = lax.rem(j, 4)
        pltpu.async_copy(ei_hbm.at[0, row0 + j], sidx.at[b4], isem)
        pltpu.async_copy(ei_hbm.at[1, row0 + j], tidx.at[b4], isem)

    def _drain_idx():
        pltpu.make_async_copy(ei_hbm.at[0, 0], sidx.at[0], isem).wait()
        pltpu.make_async_copy(ei_hbm.at[1, 0], tidx.at[0], isem).wait()

    def _fire_gathers(j):
        b2 = lax.rem(j, 2)
        b3 = lax.rem(j, 3)
        b4 = lax.rem(j, 4)
        pltpu.async_copy(proj_hbm.at[sidx.at[b4]],
                         projbuf.at[b3, :, pl.ds(0, HF)], gsem)
        pltpu.async_copy(exps_hbm.at[:, pl.ds(ebase + j * CH, CH)],
                         attnbuf.at[b2], esem)

    def _drain_gathers():
        pltpu.make_async_copy(proj_hbm.at[sidx.at[0]],
                              projbuf.at[0, :, pl.ds(0, HF)],
                              gsem).wait()
        pltpu.make_async_copy(exps_hbm.at[:, pl.ds(0, CH)], attnbuf.at[0],
                              esem).wait()

    def _drain_scatter():
        pltpu.make_async_copy(projbuf.at[0, :, pl.ds(0, HF)],
                              acc_sh.at[tidx.at[0]], ssem).wait()

    _fire_idx(0)
    _fire_idx(1)
    _drain_idx()
    _fire_gathers(0)

    def _chunk(j, carry):
        @pl.when(j >= 2)
        def _():
            _drain_scatter()

        @pl.when(j < NCHUNK - 2)
        def _():
            _fire_idx(j + 2)

        @pl.when(j < NCHUNK - 1)
        def _():
            _drain_idx()
            _fire_gathers(j + 1)

        _drain_gathers()

        b2 = lax.rem(j, 2)
        b3 = lax.rem(j, 3)
        b5 = lax.rem(j, 4)
        att = []
        for g in range(CH // L):
            row = []
            for h in range(H):
                row.append(attnbuf[b2, h, pl.ds(g * L, L)])
            att.append(row)

        pb = projbuf.at[b3]
        rows_l = [g * L + iota16 for g in range(CH // L)]

        def _col(jc, carry2):
            coff = (jnp.full((L,), jc, dtype=i32) + iota16) & (F_OUT - 1)
            for h in range(H):
                col = coff + (h * F_OUT)
                for g in range(CH // L):
                    v = plsc.load_gather(pb, [rows_l[g], col])
                    plsc.store_scatter(pb, [rows_l[g], col], v * att[g][h])
            return carry2
        lax.fori_loop(0, F_OUT, _col, 0)

        pltpu.async_copy(projbuf.at[b3, :, pl.ds(0, HF)],
                         acc_sh.at[tidx.at[b5]], ssem, add=True)
        return carry

    lax.fori_loop(0, NCHUNK, _chunk, 0)

    _drain_scatter()
    _drain_scatter()

    plsc.subcore_barrier()

    pltpu.sync_copy(acc_sh.at[pl.ds(rbase, ROWS_PER_TILE)],
                    aprt_hbm.at[c, pl.ds(rbase, ROWS_PER_TILE)])


def _k2(exps, ei3, proj, zeros):
    kfn = pl.kernel(
        _k2_body,
        out_type=jax.ShapeDtypeStruct((NC, N, HF), f32),
        mesh=plsc.VectorSubcoreMesh(core_axis_name="c", subcore_axis_name="s"),
        compiler_params=pltpu.CompilerParams(use_tc_tiling_on_sc=False,
                                             needs_layout_passes=False),
        scratch_types=[
            pltpu.VMEM((4, CH), i32),
            pltpu.VMEM((4, CH), i32),
            pltpu.VMEM((3, CH, HF), f32),
            pltpu.VMEM((2, H, CH), f32),
            pltpu.SemaphoreType.DMA,
            pltpu.SemaphoreType.DMA,
            pltpu.SemaphoreType.DMA,
            pltpu.SemaphoreType.DMA,
            pltpu.VMEM_SHARED((N, HF), f32),
        ],
    )
    return kfn(exps, ei3, proj, zeros)


def _tc2_body(aprt_ref, cprt_ref, x_ref, wpt_ref, wst_ref, bias_ref,
              oneh_ref, out_ref):
    relsum = cprt_ref[0, :, 0:HF] + cprt_ref[1, :, 0:HF]
    denom4 = (cprt_ref[0, :, HF:HF + 4] + cprt_ref[1, :, HF:HF + 4]) + 1e-16
    dfull = jnp.dot(denom4, oneh_ref[...], preferred_element_type=f32)
    v = (aprt_ref[0] + aprt_ref[1]) / dfull
    v = v + jnp.dot(relsum, wpt_ref[...], preferred_element_type=f32)
    v = v + jnp.dot(x_ref[...], wst_ref[...], preferred_element_type=f32)
    v = v + bias_ref[...]
    vm = jnp.minimum(v, 0.0)
    out_ref[...] = jnp.where(v > 0, v, jnp.exp(vm) - 1.0)


def _tc2(aprt, cprt, x, wpt, wst, bias, oneh):
    return pl.pallas_call(
        _tc2_body,
        out_shape=jax.ShapeDtypeStruct((N, HF), f32),
    )(aprt, cprt, x, wpt, wst, bias, oneh)


@jax.jit
def _run(x, ei, rel, W_proj, a_src, a_trg, W_skip, bias):
    wpt = W_proj.T
    wst = W_skip.T
    af_src = a_src.reshape(HF)
    af_trg = a_trg.reshape(HF)
    hid = jnp.arange(HF, dtype=i32) // F_OUT
    onehot = (hid[:, None] == jnp.arange(H, dtype=i32)[None, :]).astype(f32)
    sel = jnp.concatenate(
        [onehot * af_src[:, None], onehot * af_trg[:, None],
         jnp.zeros((HF, 8), dtype=f32)], axis=1)
    ei3 = ei.reshape(2, NW * NCHUNK, CH)
    zeros = jnp.zeros((ROWS_PER_TILE, CW), dtype=f32)
    bias2 = jnp.broadcast_to(bias[None, :], (8, HF))

    proj, stab, mhat = _tc1(x, wpt, sel)
    exps, cprt = _k1(stab, mhat, ei3, rel, zeros)
    aprt = _k2(exps, ei3, proj, zeros)
    out = _tc2(aprt, cprt, x, wpt, wst, bias2[0:1].reshape(HF), onehot.T)
    return out


def kernel(in_nodes_features, edge_index, rel_features, W_proj, a_src, a_trg,
           W_skip, bias):
    x = in_nodes_features[0]
    ei = edge_index[0]
    rel = rel_features[0]
    out = _run(x, ei, rel, W_proj, a_src, a_trg, W_skip, bias)
    return (out[None], edge_index, rel_features)


# speedup vs baseline: 1.0014x; 1.0014x over previous
import jax
import jax.numpy as jnp
from jax import lax
from jax.experimental import pallas as pl
from jax.experimental.pallas import tpu as pltpu
from jax.experimental.pallas import tpu_sc as plsc

N = 10000
E = 320000
F_IN = 128
H = 4
F_OUT = 32
HF = H * F_OUT
CW = HF + 16

NC = 2
NS = 16
NW = NC * NS
L = 16
EW = E // NW
CH = 80
NCHUNK = EW // CH
ROWS_PER_TILE = N // NS

f32 = jnp.float32
i32 = jnp.int32


def _full16(v):
    return jnp.full((L,), v, dtype=i32)


def _tc1_body(x_ref, wpt_ref, sel_ref, proj_ref, stab_ref, mhat_ref):
    xb = x_ref[...]
    p = jnp.dot(xb, wpt_ref[...], preferred_element_type=f32)
    proj_ref[...] = p
    st = jnp.dot(p, sel_ref[...], preferred_element_type=f32)
    stab_ref[...] = st
    m0 = jnp.max(st[:, 0:H]) + jnp.max(st[:, H:2 * H])
    mhat = jnp.maximum(m0, 0.2 * m0)
    mhat_ref[...] = jnp.full((8, 128), mhat, dtype=f32)


def _tc1(x, wpt, sel):
    return pl.pallas_call(
        _tc1_body,
        out_shape=(
            jax.ShapeDtypeStruct((N, HF), f32),
            jax.ShapeDtypeStruct((N, 16), f32),
            jax.ShapeDtypeStruct((8, 128), f32),
        ),
    )(x, wpt, sel)


def _k1_body(tab_hbm, mhat_hbm, ei_hbm, rel_hbm, zeros_hbm,
             exps_hbm, cprt_hbm,
             mhat_v, sidx, tidx, combo, sbuf, tbuf, expbuf,
             isem, gsem, rsem, esem, ssem,
             combo_sh):
    c = lax.axis_index("c")
    s = lax.axis_index("s")
    wid = c * NS + s
    ebase = wid * EW
    rbase = s * ROWS_PER_TILE
    row0 = wid * NCHUNK

    pltpu.sync_copy(zeros_hbm, combo_sh.at[pl.ds(rbase, ROWS_PER_TILE)])
    pltpu.sync_copy(mhat_hbm.at[0, pl.ds(0, L)], mhat_v)

    def _zrow(r, carry):
        for b in range(3):
            combo[b, r, pl.ds(HF, 16)] = jnp.zeros((L,), dtype=f32)
        return carry
    lax.fori_loop(0, CH, _zrow, 0)

    plsc.subcore_barrier()

    mhat_vec = mhat_v[:]
    iota16 = lax.iota(i32, L)

    def _fire_idx(j):
        b4 = lax.rem(j, 4)
        pltpu.async_copy(ei_hbm.at[0, row0 + j], sidx.at[b4], isem)
        pltpu.async_copy(ei_hbm.at[1, row0 + j], tidx.at[b4], isem)

    def _drain_idx():
        pltpu.make_async_copy(ei_hbm.at[0, 0], sidx.at[0], isem).wait()
        pltpu.make_async_copy(ei_hbm.at[1, 0], tidx.at[0], isem).wait()

    def _fire_gathers(j):
        b2 = lax.rem(j, 2)
        b3 = lax.rem(j, 3)
        b4 = lax.rem(j, 4)
        pltpu.async_copy(tab_hbm.at[sidx.at[b4]], sbuf.at[b2], gsem)
        pltpu.async_copy(tab_hbm.at[tidx.at[b4]], tbuf.at[b2], gsem)
        pltpu.async_copy(rel_hbm.at[pl.ds(ebase + j * CH, CH)],
                         combo.at[b3, :, pl.ds(0, HF)], rsem)

    def _drain_gathers():
        pltpu.make_async_copy(tab_hbm.at[sidx.at[0]], sbuf.at[0], gsem).wait()
        pltpu.make_async_copy(tab_hbm.at[tidx.at[0]], tbuf.at[0], gsem).wait()
        pltpu.make_async_copy(rel_hbm.at[pl.ds(0, CH)],
                              combo.at[0, :, pl.ds(0, HF)], rsem).wait()

    def _drain_scatter():
        pltpu.make_async_copy(combo.at[0], combo_sh.at[tidx.at[0]],
                              ssem).wait()

    def _drain_expw():
        pltpu.make_async_copy(expbuf.at[0],
                              exps_hbm.at[:, pl.ds(0, CH)], esem).wait()

    _fire_idx(0)
    _fire_idx(1)
    _drain_idx()
    _fire_gathers(0)

    def _chunk(j, carry):
        @pl.when(j >= 2)
        def _():
            _drain_scatter()
            _drain_expw()

        @pl.when(j < NCHUNK - 2)
        def _():
            _fire_idx(j + 2)

        @pl.when(j < NCHUNK - 1)
        def _():
            _drain_idx()
            _fire_gathers(j + 1)

        _drain_gathers()

        b2 = lax.rem(j, 2)
        b3 = lax.rem(j, 3)
        b5 = lax.rem(j, 4)
        for g in range(CH // L):
            rows16 = g * L + iota16
            for h in range(H):
                a = plsc.load_gather(sbuf.at[b2], [rows16, _full16(h)])
                b = plsc.load_gather(tbuf.at[b2], [rows16, _full16(H + h)])
                sc = a + b
                sc = jnp.maximum(sc, 0.2 * sc) - mhat_vec
                ex = jnp.exp(sc)
                plsc.store_scatter(combo.at[b3], [rows16, _full16(HF + h)],
                                   ex)
                expbuf[b2, h, pl.ds(g * L, L)] = ex
        pltpu.async_copy(expbuf.at[b2],
                         exps_hbm.at[:, pl.ds(ebase + j * CH, CH)], esem)
        pltpu.async_copy(combo.at[b3], combo_sh.at[tidx.at[b5]], ssem,
                         add=True)
        return carry

    lax.fori_loop(0, NCHUNK, _chunk, 0)

    _drain_scatter()
    _drain_scatter()
    _drain_expw()
    _drain_expw()

    plsc.subcore_barrier()

    pltpu.sync_copy(combo_sh.at[pl.ds(rbase, ROWS_PER_TILE)],
                    cprt_hbm.at[c, pl.ds(rbase, ROWS_PER_TILE)])


def _k1(stab, mhat, ei3, rel, zeros):
    kfn = pl.kernel(
        _k1_body,
        out_type=(
            jax.ShapeDtypeStruct((H, E), f32),
            jax.ShapeDtypeStruct((NC, N, CW), f32),
        ),
        mesh=plsc.VectorSubcoreMesh(core_axis_name="c", subcore_axis_name="s"),
        compiler_params=pltpu.CompilerParams(use_tc_tiling_on_sc=False,
                                             needs_layout_passes=False),
        scratch_types=[
            pltpu.VMEM((L,), f32),
            pltpu.VMEM((4, CH), i32),
            pltpu.VMEM((4, CH), i32),
            pltpu.VMEM((3, CH, CW), f32),
            pltpu.VMEM((2, CH, 16), f32),
            pltpu.VMEM((2, CH, 16), f32),
            pltpu.VMEM((2, H, CH), f32),
            pltpu.SemaphoreType.DMA,
            pltpu.SemaphoreType.DMA,
            pltpu.SemaphoreType.DMA,
            pltpu.SemaphoreType.DMA,
            pltpu.SemaphoreType.DMA,
            pltpu.VMEM_SHARED((N, CW), f32),
        ],
    )
    return kfn(stab, mhat, ei3, rel, zeros)


def _k2_body(exps_hbm, ei_hbm, proj_hbm, zeros_hbm,
             aprt_hbm,
             sidx, tidx, projbuf, outbuf, attnbuf,
             isem, gsem, esem, ssem,
             acc_sh):
    c = lax.axis_index("c")
    s = lax.axis_index("s")
    wid = c * NS + s
    ebase = wid * EW
    rbase = s * ROWS_PER_TILE
    row0 = wid * NCHUNK

    pltpu.sync_copy(zeros_hbm.at[:, pl.ds(0, HF)],
                    acc_sh.at[pl.ds(rbase, ROWS_PER_TILE)])

    plsc.subcore_barrier()

    iota16 = lax.iota(i32, L)

    def _fire_idx(j):
        b4 = lax.rem(j, 4)
        pltpu.async_copy(ei_hbm.at[0, row0 + j], sidx.at[b4], isem)
        pltpu.async_copy(ei_hbm.at[1, row0 + j], tidx.at[b4], isem)

    def _drain_idx():
        pltpu.make_async_copy(ei_hbm.at[0, 0], sidx.at[0], isem).wait()
        pltpu.make_async_copy(ei_hbm.at[1, 0], tidx.at[0], isem).wait()

    def _fire_gathers(j):
        b2 = lax.rem(j, 2)
        b4 = lax.rem(j, 4)
        pltpu.async_copy(proj_hbm.at[sidx.at[b4]], projbuf.at[b2], gsem)
        pltpu.async_copy(exps_hbm.at[:, pl.ds(ebase + j * CH, CH)],
                         attnbuf.at[b2], esem)

    def _drain_gathers():
        pltpu.make_async_copy(proj_hbm.at[sidx.at[0]], projbuf.at[0],
                              gsem).wait()
        pltpu.make_async_copy(exps_hbm.at[:, pl.ds(0, CH)], attnbuf.at[0],
                              esem).wait()

    def _drain_scatter():
        pltpu.make_async_copy(outbuf.at[0], acc_sh.at[tidx.at[0]],
                              ssem).wait()

    _fire_idx(0)
    _fire_idx(1)
    _drain_idx()
    _fire_gathers(0)

    def _chunk(j, carry):
        @pl.when(j >= 2)
        def _():
            _drain_scatter()

        @pl.when(j < NCHUNK - 2)
        def _():
            _fire_idx(j + 2)

        @pl.when(j < NCHUNK - 1)
        def _():
            _drain_idx()
            _fire_gathers(j + 1)

        _drain_gathers()

        b2 = lax.rem(j, 2)
        b5 = lax.rem(j, 4)
        att = []
        for g in range(CH // L):
            row = []
            for h in range(H):
                row.append(attnbuf[b2, h, pl.ds(g * L, L)])
            att.append(row)

        pb = projbuf.at[b2]
        ob = outbuf.at[b2]
        rows_l = [g * L + iota16 for g in range(CH // L)]

        def _col(jc, carry2):
            coff = (jnp.full((L,), jc, dtype=i32) + iota16) & (F_OUT - 1)
            for h in range(H):
                col = coff + (h * F_OUT)
                for g in range(CH // L):
                    v = plsc.load_gather(pb, [rows_l[g], col])
                    plsc.store_scatter(ob, [rows_l[g], col], v * att[g][h])
            return carry2
        lax.fori_loop(0, F_OUT, _col, 0)

        pltpu.async_copy(outbuf.at[b2], acc_sh.at[tidx.at[b5]], ssem,
                         add=True)
        return carry

    lax.fori_loop(0, NCHUNK, _chunk, 0)

    _drain_scatter()
    _drain_scatter()

    plsc.subcore_barrier()

    pltpu.sync_copy(acc_sh.at[pl.ds(rbase, ROWS_PER_TILE)],
                    aprt_hbm.at[c, pl.ds(rbase, ROWS_PER_TILE)])


def _k2(exps, ei3, proj, zeros):
    kfn = pl.kernel(
        _k2_body,
        out_type=jax.ShapeDtypeStruct((NC, N, HF), f32),
        mesh=plsc.VectorSubcoreMesh(core_axis_name="c", subcore_axis_name="s"),
        compiler_params=pltpu.CompilerParams(use_tc_tiling_on_sc=False,
                                             needs_layout_passes=False),
        scratch_types=[
            pltpu.VMEM((4, CH), i32),
            pltpu.VMEM((4, CH), i32),
            pltpu.VMEM((2, CH, HF), f32),
            pltpu.VMEM((2, CH, HF), f32),
            pltpu.VMEM((2, H, CH), f32),
            pltpu.SemaphoreType.DMA,
            pltpu.SemaphoreType.DMA,
            pltpu.SemaphoreType.DMA,
            pltpu.SemaphoreType.DMA,
            pltpu.VMEM_SHARED((N, HF), f32),
        ],
    )
    return kfn(exps, ei3, proj, zeros)


def _tc2_body(aprt_ref, cprt_ref, x_ref, wpt_ref, wst_ref, bias_ref,
              oneh_ref, out_ref):
    relsum = cprt_ref[0, :, 0:HF] + cprt_ref[1, :, 0:HF]
    denom4 = (cprt_ref[0, :, HF:HF + 4] + cprt_ref[1, :, HF:HF + 4]) + 1e-16
    dfull = jnp.dot(denom4, oneh_ref[...], preferred_element_type=f32)
    v = (aprt_ref[0] + aprt_ref[1]) / dfull
    v = v + jnp.dot(relsum, wpt_ref[...], preferred_element_type=f32)
    v = v + jnp.dot(x_ref[...], wst_ref[...], preferred_element_type=f32)
    v = v + bias_ref[...]
    vm = jnp.minimum(v, 0.0)
    out_ref[...] = jnp.where(v > 0, v, jnp.exp(vm) - 1.0)


def _tc2(aprt, cprt, x, wpt, wst, bias, oneh):
    return pl.pallas_call(
        _tc2_body,
        out_shape=jax.ShapeDtypeStruct((N, HF), f32),
    )(aprt, cprt, x, wpt, wst, bias, oneh)


@jax.jit
def _run(x, ei, rel, W_proj, a_src, a_trg, W_skip, bias):
    wpt = W_proj.T
    wst = W_skip.T
    af_src = a_src.reshape(HF)
    af_trg = a_trg.reshape(HF)
    hid = jnp.arange(HF, dtype=i32) // F_OUT
    onehot = (hid[:, None] == jnp.arange(H, dtype=i32)[None, :]).astype(f32)
    sel = jnp.concatenate(
        [onehot * af_src[:, None], onehot * af_trg[:, None],
         jnp.zeros((HF, 8), dtype=f32)], axis=1)
    ei3 = ei.reshape(2, NW * NCHUNK, CH)
    zeros = jnp.zeros((ROWS_PER_TILE, CW), dtype=f32)
    bias2 = jnp.broadcast_to(bias[None, :], (8, HF))

    proj, stab, mhat = _tc1(x, wpt, sel)
    exps, cprt = _k1(stab, mhat, ei3, rel, zeros)
    aprt = _k2(exps, ei3, proj, zeros)
    out = _tc2(aprt, cprt, x, wpt, wst, bias2[0:1].reshape(HF), onehot.T)
    return out


def kernel(in_nodes_features, edge_index, rel_features, W_proj, a_src, a_trg,
           W_skip, bias):
    x = in_nodes_features[0]
    ei = edge_index[0]
    rel = rel_features[0]
    out = _run(x, ei, rel, W_proj, a_src, a_trg, W_skip, bias)
    return (out[None], edge_index, rel_features)
